# Initial kernel scaffold; baseline (speedup 1.0000x reference)
import jax
import jax.numpy as jnp
from jax.experimental import pallas as pl


def kernel(x, edge_index, edge_weight, h0, alpha, beta, W1, W2):
    raise NotImplementedError("write your pallas kernel here")


# speedup vs baseline: 4.2833x; 4.2833x over previous
import functools

import jax
import jax.numpy as jnp
from jax import lax
from jax.experimental import pallas as pl
from jax.experimental.pallas import tpu as pltpu
from jax.experimental.pallas import tpu_sc as plsc

N_NODES = 10000
D = 128
NC = 2
NS = 16
NW = NC * NS
EDGE_BLK = 128
ROWS_PER_SUB = 624
TAIL_ROWS = N_NODES - NS * ROWS_PER_SUB


def _sc_aggregate(x, src, dst, w, zeros):
  ch = src.shape[1]
  mesh = plsc.VectorSubcoreMesh(core_axis_name="c", subcore_axis_name="s")

  @functools.partial(
      pl.kernel,
      mesh=mesh,
      out_type=jax.ShapeDtypeStruct((NC, N_NODES, D), jnp.float32),
      scratch_types=[
          pltpu.VMEM((ch, EDGE_BLK), jnp.int32),
          pltpu.VMEM((ch, EDGE_BLK), jnp.int32),
          pltpu.VMEM((ch, EDGE_BLK), jnp.float32),
          pltpu.VMEM((EDGE_BLK, D), jnp.float32),
          pltpu.VMEM_SHARED((N_NODES, D), jnp.float32),
          pltpu.SemaphoreType.DMA,
      ],
  )
  def k(x_hbm, src_hbm, dst_hbm, w_hbm, z_hbm, out_hbm,
        src_v, dst_v, w_v, rows_v, acc, sem):
    c = lax.axis_index("c")
    s = lax.axis_index("s")
    wid = c * NS + s

    pltpu.sync_copy(z_hbm.at[pl.ds(s * ROWS_PER_SUB, ROWS_PER_SUB)],
                    acc.at[pl.ds(s * ROWS_PER_SUB, ROWS_PER_SUB)])

    @pl.when(s == NS - 1)
    def _zero_tail():
      pltpu.sync_copy(z_hbm.at[pl.ds(NS * ROWS_PER_SUB, TAIL_ROWS)],
                      acc.at[pl.ds(NS * ROWS_PER_SUB, TAIL_ROWS)])

    pltpu.sync_copy(src_hbm.at[wid], src_v)
    pltpu.sync_copy(dst_hbm.at[wid], dst_v)
    pltpu.sync_copy(w_hbm.at[wid], w_v)

    plsc.subcore_barrier()

    def chunk_body(j, carry):
      pltpu.async_copy(x_hbm.at[src_v.at[j]], rows_v, sem).wait()

      def group_body(g, carry2):
        base = g * 16
        wvec = w_v[j, pl.ds(base, 16)]
        for l in range(16):
          wt = wvec[l]
          i = base + l
          for f in range(D // 16):
            sl = pl.ds(f * 16, 16)
            rows_v[i, sl] = rows_v[i, sl] * wt
        return carry2

      lax.fori_loop(0, EDGE_BLK // 16, group_body, 0)

      pltpu.sync_copy(rows_v, acc.at[dst_v.at[j]], add=True)
      return carry

    lax.fori_loop(0, ch, chunk_body, 0)

    plsc.subcore_barrier()

    pltpu.sync_copy(acc.at[pl.ds(s * ROWS_PER_SUB, ROWS_PER_SUB)],
                    out_hbm.at[c, pl.ds(s * ROWS_PER_SUB, ROWS_PER_SUB)])

    @pl.when(s == NS - 1)
    def _write_tail():
      pltpu.sync_copy(acc.at[pl.ds(NS * ROWS_PER_SUB, TAIL_ROWS)],
                      out_hbm.at[c, pl.ds(NS * ROWS_PER_SUB, TAIL_ROWS)])

  return k(x, src, dst, w, zeros)


def _tc_combine_kernel(ab_ref, p0_ref, p1_ref, h0_ref, w1_ref, w2_ref, out_ref):
  a = ab_ref[0]
  b = ab_ref[1]
  eye = (lax.broadcasted_iota(jnp.int32, (D, D), 0)
         == lax.broadcasted_iota(jnp.int32, (D, D), 1)).astype(jnp.float32)
  m1 = (1.0 - b) * eye + b * w1_ref[...]
  m2 = (1.0 - b) * eye + b * w2_ref[...]
  agg = p0_ref[...] + p1_ref[...]
  left = jnp.dot(agg, m1, preferred_element_type=jnp.float32)
  right = jnp.dot(h0_ref[...], m2, preferred_element_type=jnp.float32)
  out_ref[...] = (1.0 - a) * left + a * right


def _tc_combine(partials, h0, w1, w2, alpha, beta):
  blk = 1000
  grid = N_NODES // blk
  ab = jnp.stack([alpha, beta]).astype(jnp.float32)
  return pl.pallas_call(
      _tc_combine_kernel,
      grid=(grid,),
      in_specs=[
          pl.BlockSpec(memory_space=pltpu.SMEM),
          pl.BlockSpec((blk, D), lambda i: (i, 0)),
          pl.BlockSpec((blk, D), lambda i: (i, 0)),
          pl.BlockSpec((blk, D), lambda i: (i, 0)),
          pl.BlockSpec((D, D), lambda i: (0, 0)),
          pl.BlockSpec((D, D), lambda i: (0, 0)),
      ],
      out_specs=pl.BlockSpec((blk, D), lambda i: (i, 0)),
      out_shape=jax.ShapeDtypeStruct((N_NODES, D), jnp.float32),
  )(ab, partials[0], partials[1], h0, w1, w2)


def kernel(x, edge_index, edge_weight, h0, alpha, beta, W1, W2):
  n_edges = edge_index.shape[1]
  per_worker = -(-n_edges // (NW * EDGE_BLK)) * EDGE_BLK
  pad = NW * per_worker - n_edges

  src = edge_index[0].astype(jnp.int32)
  dst = edge_index[1].astype(jnp.int32)
  w = edge_weight.astype(jnp.float32)
  if pad:
    src = jnp.concatenate([src, jnp.zeros((pad,), jnp.int32)])
    dst = jnp.concatenate([dst, jnp.zeros((pad,), jnp.int32)])
    w = jnp.concatenate([w, jnp.zeros((pad,), jnp.float32)])
  ch = per_worker // EDGE_BLK
  src = src.reshape(NW, ch, EDGE_BLK)
  dst = dst.reshape(NW, ch, EDGE_BLK)
  w = w.reshape(NW, ch, EDGE_BLK)

  zeros = jnp.zeros((N_NODES, D), jnp.float32)
  partials = _sc_aggregate(x, src, dst, w, zeros)
  return _tc_combine(partials, h0, W1, W2, alpha, beta)
